# Initial kernel scaffold; baseline (speedup 1.0000x reference)
import jax
import jax.numpy as jnp
from jax.experimental import pallas as pl


def kernel(x, edge_index, W, b, a):
    raise NotImplementedError("write your pallas kernel here")


# speedup vs baseline: 18.0950x; 18.0950x over previous
import functools

import jax
import jax.numpy as jnp
from jax import lax
from jax.experimental import pallas as pl
from jax.experimental.pallas import tpu as pltpu
from jax.experimental.pallas import tpu_sc as plsc

NC = 2
NS = 16
NW = NC * NS

_MESH = functools.partial(
    plsc.VectorSubcoreMesh, core_axis_name="c", subcore_axis_name="s"
)


def _deg_kernel(E, NPAD, K):
    EPT = E // NW
    NCHUNK = EPT // K
    PT = NPAD // NS

    @functools.partial(
        pl.kernel,
        out_type=jax.ShapeDtypeStruct((NC * NPAD,), jnp.float32),
        mesh=_MESH(),
        scratch_types=[
            pltpu.VMEM((PT,), jnp.float32),
            pltpu.VMEM((K,), jnp.float32),
            pltpu.VMEM((K,), jnp.int32),
            pltpu.VMEM_SHARED((NPAD,), jnp.float32),
        ],
    )
    def deg_kernel(dst_hbm, deg_out, zbuf, ones_v, idx_v, hist):
        c = lax.axis_index("c")
        s = lax.axis_index("s")

        def fill_z(i, _):
            zbuf[pl.ds(i * 16, 16)] = jnp.zeros((16,), jnp.float32)
            return 0

        lax.fori_loop(0, PT // 16, fill_z, 0)

        def fill_o(i, _):
            ones_v[pl.ds(i * 16, 16)] = jnp.ones((16,), jnp.float32)
            return 0

        lax.fori_loop(0, K // 16, fill_o, 0)

        pltpu.sync_copy(zbuf, hist.at[pl.ds(s * PT, PT)])
        plsc.subcore_barrier()

        base = (c * NS + s) * EPT

        def body(i, _):
            pltpu.sync_copy(dst_hbm.at[pl.ds(base + i * K, K)], idx_v)
            pltpu.sync_copy(ones_v, hist.at[idx_v], add=True)
            return 0

        lax.fori_loop(0, NCHUNK, body, 0)
        plsc.subcore_barrier()
        pltpu.sync_copy(
            hist.at[pl.ds(s * PT, PT)],
            deg_out.at[pl.ds(c * NPAD + s * PT, PT)],
        )

    return deg_kernel


def _agg_kernel(N, E, D, K):
    EPT = E // NW
    NCHUNK = EPT // K
    RPT = N // NS
    CNT = ((RPT + 7) // 8) * 8
    NZ, ZREM = RPT // K, RPT % K

    @functools.partial(
        pl.kernel,
        out_type=jax.ShapeDtypeStruct((NC * N, D), jnp.float32),
        mesh=_MESH(),
        scratch_types=[
            pltpu.VMEM((K, D), jnp.float32),
            pltpu.VMEM((K,), jnp.int32),
            pltpu.VMEM((K,), jnp.int32),
            pltpu.VMEM_SHARED((N, D), jnp.float32),
            pltpu.SemaphoreType.DMA,
        ],
    )
    def agg_kernel(g_hbm, src_hbm, dst_hbm, s_out, rows_v, sidx, didx, acc, sem):
        c = lax.axis_index("c")
        s = lax.axis_index("s")

        def fill_z(i, _):
            for j in range(D // 16):
                rows_v[i, pl.ds(j * 16, 16)] = jnp.zeros((16,), jnp.float32)
            return 0

        lax.fori_loop(0, K, fill_z, 0)

        row0 = s * RPT
        for k in range(NZ):
            pltpu.sync_copy(rows_v, acc.at[pl.ds(row0 + k * K, K)])
        if ZREM:
            pltpu.sync_copy(
                rows_v.at[pl.ds(0, ZREM)], acc.at[pl.ds(row0 + NZ * K, ZREM)]
            )
        plsc.subcore_barrier()

        base = (c * NS + s) * EPT

        def body(i, _):
            off = base + i * K
            pltpu.sync_copy(src_hbm.at[pl.ds(off, K)], sidx)
            pltpu.sync_copy(dst_hbm.at[pl.ds(off, K)], didx)
            pltpu.async_copy(g_hbm.at[sidx], rows_v, sem).wait()
            pltpu.sync_copy(rows_v, acc.at[didx], add=True)
            return 0

        lax.fori_loop(0, NCHUNK, body, 0)
        plsc.subcore_barrier()

        bs = pl.multiple_of((s * RPT // 8) * 8, 8)
        bs = jnp.minimum(bs, N - CNT)
        pltpu.sync_copy(
            acc.at[pl.ds(bs, CNT)], s_out.at[pl.ds(c * N + bs, CNT)]
        )

    return agg_kernel


def _dinv_body(deg_ref, out_ref):
    d = deg_ref[0:1, :] + deg_ref[1:2, :] + 1.0
    out_ref[...] = lax.rsqrt(d)


def _g_body(x_ref, w_ref, dinv_ref, g_ref):
    h = lax.dot_general(
        x_ref[...], w_ref[...], (((1,), (1,)), ((), ())),
        preferred_element_type=jnp.float32,
    )
    g_ref[...] = dinv_ref[...] * h


def _fin_body(s0_ref, s1_ref, g_ref, dinv_ref, b_ref, a_ref, o_ref):
    t = s0_ref[...] + s1_ref[...] + g_ref[...]
    t = dinv_ref[...] * t + b_ref[...]
    o_ref[...] = jnp.where(t >= 0.0, t, a_ref[0, 0] * t)


def kernel(x, edge_index, W, b, a):
    N, D = x.shape
    E = edge_index.shape[1]
    K = 80
    NPAD = ((N + NS * 16 - 1) // (NS * 16)) * (NS * 16)

    src = edge_index[0]
    dst = edge_index[1]

    deg_flat = _deg_kernel(E, NPAD, K)(dst)
    dinv_row = pl.pallas_call(
        _dinv_body,
        out_shape=jax.ShapeDtypeStruct((1, NPAD), jnp.float32),
    )(deg_flat.reshape(NC, NPAD))
    dinv_col = dinv_row.reshape(NPAD, 1)[:N]

    RB = 2000
    g = pl.pallas_call(
        _g_body,
        grid=(N // RB,),
        in_specs=[
            pl.BlockSpec((RB, D), lambda i: (i, 0)),
            pl.BlockSpec((D, D), lambda i: (0, 0)),
            pl.BlockSpec((RB, 1), lambda i: (i, 0)),
        ],
        out_specs=pl.BlockSpec((RB, D), lambda i: (i, 0)),
        out_shape=jax.ShapeDtypeStruct((N, D), jnp.float32),
    )(x, W, dinv_col)

    s_flat = _agg_kernel(N, E, D, K)(g, src, dst)

    nb = N // RB
    out = pl.pallas_call(
        _fin_body,
        grid=(nb,),
        in_specs=[
            pl.BlockSpec((RB, D), lambda i: (i, 0)),
            pl.BlockSpec((RB, D), lambda i, nb=nb: (i + nb, 0)),
            pl.BlockSpec((RB, D), lambda i: (i, 0)),
            pl.BlockSpec((RB, 1), lambda i: (i, 0)),
            pl.BlockSpec((1, D), lambda i: (0, 0)),
            pl.BlockSpec((1, 1), lambda i: (0, 0)),
        ],
        out_specs=pl.BlockSpec((RB, D), lambda i: (i, 0)),
        out_shape=jax.ShapeDtypeStruct((N, D), jnp.float32),
    )(s_flat, s_flat, g, dinv_col, jnp.reshape(b, (1, D)), jnp.reshape(a, (1, 1)))
    return out


# speedup vs baseline: 44.9096x; 2.4819x over previous
import functools

import jax
import jax.numpy as jnp
from jax import lax
from jax.experimental import pallas as pl
from jax.experimental.pallas import tpu as pltpu
from jax.experimental.pallas import tpu_sc as plsc

NC = 2
NS = 16
NW = NC * NS

_MESH = functools.partial(
    plsc.VectorSubcoreMesh, core_axis_name="c", subcore_axis_name="s"
)


def _deg_kernel(E, NPAD, K, NCHUNK):
    PT = NPAD // NS
    G = 10
    assert NCHUNK % G == 0
    NGRP = NCHUNK // G

    @functools.partial(
        pl.kernel,
        out_type=jax.ShapeDtypeStruct((NC * NPAD,), jnp.float32),
        mesh=_MESH(),
        scratch_types=[
            pltpu.VMEM((PT,), jnp.float32),
            pltpu.VMEM((128,), jnp.float32),
            pltpu.VMEM((NCHUNK, K), jnp.int32),
            pltpu.VMEM_SHARED((NPAD,), jnp.float32),
            pltpu.SemaphoreType.DMA,
        ],
    )
    def deg_kernel(dst3_hbm, deg_out, zbuf, ones_v, idx_v, hist, sem):
        c = lax.axis_index("c")
        s = lax.axis_index("s")
        w = c * NS + s

        def fill_z(i, _):
            zbuf[pl.ds(i * 16, 16)] = jnp.zeros((16,), jnp.float32)
            return 0

        lax.fori_loop(0, PT // 16, fill_z, 0)
        for j in range(8):
            ones_v[pl.ds(j * 16, 16)] = jnp.ones((16,), jnp.float32)

        pltpu.sync_copy(dst3_hbm.at[w], idx_v)
        pltpu.sync_copy(zbuf, hist.at[pl.ds(s * PT, PT)])
        plsc.subcore_barrier()

        ones_k = ones_v.at[pl.ds(0, K)]

        def body(gi, _):
            for b in range(G):
                pltpu.async_copy(
                    ones_k, hist.at[idx_v.at[gi * G + b]], sem, add=True
                )
            for b in range(G):
                pltpu.make_async_copy(
                    ones_k, hist.at[idx_v.at[gi * G + b]], sem
                ).wait()
            return 0

        lax.fori_loop(0, NGRP, body, 0)
        plsc.subcore_barrier()
        pltpu.sync_copy(
            hist.at[pl.ds(s * PT, PT)],
            deg_out.at[pl.ds(c * NPAD + s * PT, PT)],
        )

    return deg_kernel


def _agg_kernel(N, E, D, K, NCHUNK):
    RPT = N // NS
    CNT = ((RPT + 7) // 8) * 8
    NZ, ZREM = RPT // K, RPT % K
    NBUF = 2
    NPH = 2
    HCH = NCHUNK // NPH
    NGRP = HCH // NBUF

    @functools.partial(
        pl.kernel,
        out_type=jax.ShapeDtypeStruct((NC * N, D), jnp.float32),
        mesh=_MESH(),
        scratch_types=[
            pltpu.VMEM((HCH, K), jnp.int32),
            pltpu.VMEM((HCH, K), jnp.int32),
            pltpu.VMEM((K, D), jnp.float32),
            pltpu.VMEM((K, D), jnp.float32),
            pltpu.VMEM_SHARED((N, D), jnp.float32),
            pltpu.SemaphoreType.DMA,
            pltpu.SemaphoreType.DMA,
        ],
    )
    def agg_kernel(g_hbm, src3_hbm, dst3_hbm, s_out,
                   sidx, didx, r0, r1, acc, m0, m1):
        c = lax.axis_index("c")
        s = lax.axis_index("s")
        w = c * NS + s
        rows = (r0, r1)
        sems = (m0, m1)

        def fill_z(i, _):
            for j in range(D // 16):
                r0[i, pl.ds(j * 16, 16)] = jnp.zeros((16,), jnp.float32)
            return 0

        lax.fori_loop(0, K, fill_z, 0)

        row0 = s * RPT
        for k in range(NZ):
            pltpu.sync_copy(r0, acc.at[pl.ds(row0 + k * K, K)])
        if ZREM:
            pltpu.sync_copy(
                r0.at[pl.ds(0, ZREM)], acc.at[pl.ds(row0 + NZ * K, ZREM)]
            )
        plsc.subcore_barrier()

        for ph in range(NPH):
            pltpu.sync_copy(src3_hbm.at[w, pl.ds(ph * HCH, HCH)], sidx)
            pltpu.sync_copy(dst3_hbm.at[w, pl.ds(ph * HCH, HCH)], didx)

            for b in range(NBUF):
                pltpu.async_copy(g_hbm.at[sidx.at[b]], rows[b], sems[b])

            def body(gi, _):
                for b in range(NBUF):
                    i = gi * NBUF + b
                    pltpu.make_async_copy(
                        g_hbm.at[sidx.at[i]], rows[b], sems[b]
                    ).wait()
                    pltpu.sync_copy(rows[b], acc.at[didx.at[i]], add=True)
                    pltpu.async_copy(
                        g_hbm.at[sidx.at[i + NBUF]], rows[b], sems[b]
                    )
                return 0

            lax.fori_loop(0, NGRP - 1, body, 0)
            for b in range(NBUF):
                i = HCH - NBUF + b
                pltpu.make_async_copy(
                    g_hbm.at[sidx.at[i]], rows[b], sems[b]
                ).wait()
                pltpu.sync_copy(rows[b], acc.at[didx.at[i]], add=True)

        plsc.subcore_barrier()

        bs = pl.multiple_of((s * RPT // 8) * 8, 8)
        bs = jnp.minimum(bs, N - CNT)
        pltpu.sync_copy(
            acc.at[pl.ds(bs, CNT)], s_out.at[pl.ds(c * N + bs, CNT)]
        )

    return agg_kernel


def _dinv_body(deg_ref, out_ref):
    d = deg_ref[0:1, :] + deg_ref[1:2, :] + 1.0
    out_ref[...] = lax.rsqrt(d)


def _g_body(x_ref, w_ref, dinv_ref, g_ref):
    h = lax.dot_general(
        x_ref[...], w_ref[...], (((1,), (1,)), ((), ())),
        preferred_element_type=jnp.float32,
    )
    g_ref[...] = dinv_ref[...] * h


def _fin_body(s0_ref, s1_ref, g_ref, dinv_ref, b_ref, a_ref, o_ref):
    t = s0_ref[...] + s1_ref[...] + g_ref[...]
    t = dinv_ref[...] * t + b_ref[...]
    o_ref[...] = jnp.where(t >= 0.0, t, a_ref[0, 0] * t)


def kernel(x, edge_index, W, b, a):
    N, D = x.shape
    E = edge_index.shape[1]
    K = 125
    EPT = E // NW
    NCHUNK = EPT // K
    NPAD = ((N + NS * 16 - 1) // (NS * 16)) * (NS * 16)

    src3 = edge_index[0].reshape(NW, NCHUNK, K)
    dst3 = edge_index[1].reshape(NW, NCHUNK, K)

    deg_flat = _deg_kernel(E, NPAD, K, NCHUNK)(dst3)
    dinv_row = pl.pallas_call(
        _dinv_body,
        out_shape=jax.ShapeDtypeStruct((1, NPAD), jnp.float32),
    )(deg_flat.reshape(NC, NPAD))
    dinv_col = dinv_row.reshape(NPAD, 1)[:N]

    RB = 2000
    g = pl.pallas_call(
        _g_body,
        grid=(N // RB,),
        in_specs=[
            pl.BlockSpec((RB, D), lambda i: (i, 0)),
            pl.BlockSpec((D, D), lambda i: (0, 0)),
            pl.BlockSpec((RB, 1), lambda i: (i, 0)),
        ],
        out_specs=pl.BlockSpec((RB, D), lambda i: (i, 0)),
        out_shape=jax.ShapeDtypeStruct((N, D), jnp.float32),
    )(x, W, dinv_col)

    s_flat = _agg_kernel(N, E, D, K, NCHUNK)(g, src3, dst3)

    nb = N // RB
    out = pl.pallas_call(
        _fin_body,
        grid=(nb,),
        in_specs=[
            pl.BlockSpec((RB, D), lambda i: (i, 0)),
            pl.BlockSpec((RB, D), lambda i, nb=nb: (i + nb, 0)),
            pl.BlockSpec((RB, D), lambda i: (i, 0)),
            pl.BlockSpec((RB, 1), lambda i: (i, 0)),
            pl.BlockSpec((1, D), lambda i: (0, 0)),
            pl.BlockSpec((1, 1), lambda i: (0, 0)),
        ],
        out_specs=pl.BlockSpec((RB, D), lambda i: (i, 0)),
        out_shape=jax.ShapeDtypeStruct((N, D), jnp.float32),
    )(s_flat, s_flat, g, dinv_col, jnp.reshape(b, (1, D)), jnp.reshape(a, (1, 1)))
    return out


# speedup vs baseline: 46.7341x; 1.0406x over previous
import functools

import jax
import jax.numpy as jnp
from jax import lax
from jax.experimental import pallas as pl
from jax.experimental.pallas import tpu as pltpu
from jax.experimental.pallas import tpu_sc as plsc

NC = 2
NS = 16
NW = NC * NS

_MESH = functools.partial(
    plsc.VectorSubcoreMesh, core_axis_name="c", subcore_axis_name="s"
)


def _deg_kernel(E, NPAD, K, NCHUNK):
    PT = NPAD // NS
    G = 10
    assert NCHUNK % G == 0
    NGRP = NCHUNK // G

    @functools.partial(
        pl.kernel,
        out_type=jax.ShapeDtypeStruct((NC * NPAD,), jnp.float32),
        mesh=_MESH(),
        scratch_types=[
            pltpu.VMEM((PT,), jnp.float32),
            pltpu.VMEM((128,), jnp.float32),
            pltpu.VMEM((NCHUNK, K), jnp.int32),
            pltpu.VMEM_SHARED((NPAD,), jnp.float32),
            pltpu.SemaphoreType.DMA,
        ],
    )
    def deg_kernel(dst3_hbm, deg_out, zbuf, ones_v, idx_v, hist, sem):
        c = lax.axis_index("c")
        s = lax.axis_index("s")
        w = c * NS + s

        def fill_z(i, _):
            zbuf[pl.ds(i * 16, 16)] = jnp.zeros((16,), jnp.float32)
            return 0

        lax.fori_loop(0, PT // 16, fill_z, 0)
        for j in range(8):
            ones_v[pl.ds(j * 16, 16)] = jnp.ones((16,), jnp.float32)

        pltpu.sync_copy(dst3_hbm.at[w], idx_v)
        pltpu.sync_copy(zbuf, hist.at[pl.ds(s * PT, PT)])
        plsc.subcore_barrier()

        ones_k = ones_v.at[pl.ds(0, K)]

        def body(gi, _):
            for b in range(G):
                pltpu.async_copy(
                    ones_k, hist.at[idx_v.at[gi * G + b]], sem, add=True
                )
            for b in range(G):
                pltpu.make_async_copy(
                    ones_k, hist.at[idx_v.at[gi * G + b]], sem
                ).wait()
            return 0

        lax.fori_loop(0, NGRP, body, 0)
        plsc.subcore_barrier()
        pltpu.sync_copy(
            hist.at[pl.ds(s * PT, PT)],
            deg_out.at[pl.ds(c * NPAD + s * PT, PT)],
        )

    return deg_kernel


def _agg_kernel(N, NPAD, E, D, K, NCHUNK):
    RPT = N // NS
    CNT = ((RPT + 7) // 8) * 8
    NZ, ZREM = RPT // K, RPT % K
    NBUF = 2
    NPH = 2
    HCH = NCHUNK // NPH
    NGRP = HCH // NBUF

    EPT = E // NW

    @functools.partial(
        pl.kernel,
        out_type=jax.ShapeDtypeStruct((NC * NPAD, D), jnp.float32),
        mesh=_MESH(),
        scratch_types=[
            pltpu.VMEM((HCH, K), jnp.int32),
            pltpu.VMEM((HCH, K), jnp.int32),
            pltpu.VMEM((K, D), jnp.float32),
            pltpu.VMEM((K, D), jnp.float32),
            pltpu.VMEM_SHARED((N, D), jnp.float32),
            pltpu.SemaphoreType.DMA,
            pltpu.SemaphoreType.DMA,
        ],
    )
    def agg_kernel(g_hbm, src3_hbm, dst3_hbm, s_out,
                   sidx, didx, r0, r1, acc, m0, m1):
        c = lax.axis_index("c")
        s = lax.axis_index("s")
        w = c * NS + s
        rows = (r0, r1)
        sems = (m0, m1)

        def fill_z(i, _):
            for j in range(D // 16):
                r0[i, pl.ds(j * 16, 16)] = jnp.zeros((16,), jnp.float32)
            return 0

        lax.fori_loop(0, K, fill_z, 0)

        row0 = s * RPT
        for k in range(NZ):
            pltpu.sync_copy(r0, acc.at[pl.ds(row0 + k * K, K)])
        if ZREM:
            pltpu.sync_copy(
                r0.at[pl.ds(0, ZREM)], acc.at[pl.ds(row0 + NZ * K, ZREM)]
            )
        plsc.subcore_barrier()

        for ph in range(NPH):
            pltpu.sync_copy(src3_hbm.at[w, pl.ds(ph * HCH, HCH)], sidx)
            pltpu.sync_copy(dst3_hbm.at[w, pl.ds(ph * HCH, HCH)], didx)

            for b in range(NBUF):
                pltpu.async_copy(g_hbm.at[sidx.at[b]], rows[b], sems[b])

            def body(gi, _):
                for b in range(NBUF):
                    i = gi * NBUF + b
                    pltpu.make_async_copy(
                        g_hbm.at[sidx.at[i]], rows[b], sems[b]
                    ).wait()
                    pltpu.sync_copy(rows[b], acc.at[didx.at[i]], add=True)
                    pltpu.async_copy(
                        g_hbm.at[sidx.at[i + NBUF]], rows[b], sems[b]
                    )
                return 0

            lax.fori_loop(0, NGRP - 1, body, 0)
            for b in range(NBUF):
                i = HCH - NBUF + b
                pltpu.make_async_copy(
                    g_hbm.at[sidx.at[i]], rows[b], sems[b]
                ).wait()
                pltpu.sync_copy(rows[b], acc.at[didx.at[i]], add=True)

        plsc.subcore_barrier()

        bs = pl.multiple_of((s * RPT // 8) * 8, 8)
        bs = jnp.minimum(bs, N - CNT)
        pltpu.sync_copy(
            acc.at[pl.ds(bs, CNT)], s_out.at[pl.ds(c * NPAD + bs, CNT)]
        )

    return agg_kernel


def _dinv_col(deg_ref):
    d = deg_ref[0:1, :] + deg_ref[1:2, :] + 1.0
    return jnp.transpose(lax.rsqrt(d), (1, 0))


def _g_body(x_ref, w_ref, deg_ref, g_ref):
    h = lax.dot_general(
        x_ref[...], w_ref[...], (((1,), (1,)), ((), ())),
        preferred_element_type=jnp.float32,
    )
    g_ref[...] = _dinv_col(deg_ref) * h


def _fin_body(s0_ref, s1_ref, g_ref, deg_ref, b_ref, a_ref, o_ref):
    t = s0_ref[...] + s1_ref[...] + g_ref[...]
    t = _dinv_col(deg_ref) * t + b_ref[...]
    o_ref[...] = jnp.where(t >= 0.0, t, a_ref[0, 0] * t)


def kernel(x, edge_index, W, b, a):
    N, D = x.shape
    E = edge_index.shape[1]
    K = 125
    EPT = E // NW
    NCHUNK = EPT // K
    NPAD = ((N + NS * 16 - 1) // (NS * 16)) * (NS * 16)

    src3 = edge_index[0].reshape(NW, NCHUNK, K)
    dst3 = edge_index[1].reshape(NW, NCHUNK, K)

    deg_flat = _deg_kernel(E, NPAD, K, NCHUNK)(dst3)
    deg2 = deg_flat.reshape(NC, NPAD)

    RB = 2048
    nb = NPAD // RB
    g = pl.pallas_call(
        _g_body,
        grid=(nb,),
        in_specs=[
            pl.BlockSpec((RB, D), lambda i: (i, 0)),
            pl.BlockSpec((D, D), lambda i: (0, 0)),
            pl.BlockSpec((NC, RB), lambda i: (0, i)),
        ],
        out_specs=pl.BlockSpec((RB, D), lambda i: (i, 0)),
        out_shape=jax.ShapeDtypeStruct((NPAD, D), jnp.float32),
    )(x, W, deg2)

    s_flat = _agg_kernel(N, NPAD, E, D, K, NCHUNK)(g, src3, dst3)

    out = pl.pallas_call(
        _fin_body,
        grid=(nb,),
        in_specs=[
            pl.BlockSpec((RB, D), lambda i: (i, 0)),
            pl.BlockSpec((RB, D), lambda i, nb=nb: (i + nb, 0)),
            pl.BlockSpec((RB, D), lambda i: (i, 0)),
            pl.BlockSpec((NC, RB), lambda i: (0, i)),
            pl.BlockSpec((1, D), lambda i: (0, 0)),
            pl.BlockSpec((1, 1), lambda i: (0, 0)),
        ],
        out_specs=pl.BlockSpec((RB, D), lambda i: (i, 0)),
        out_shape=jax.ShapeDtypeStruct((N, D), jnp.float32),
    )(s_flat, s_flat, g, deg2, jnp.reshape(b, (1, D)), jnp.reshape(a, (1, 1)))
    return out


# speedup vs baseline: 49.3990x; 1.0570x over previous
import functools

import jax
import jax.numpy as jnp
from jax import lax
from jax.experimental import pallas as pl
from jax.experimental.pallas import tpu as pltpu
from jax.experimental.pallas import tpu_sc as plsc

NC = 2
NS = 16
NW = NC * NS

_MESH = functools.partial(
    plsc.VectorSubcoreMesh, core_axis_name="c", subcore_axis_name="s"
)


def _deg_kernel(E, NPAD, K, NCHUNK):
    PT = NPAD // NS
    G = 10
    assert NCHUNK % G == 0
    NGRP = NCHUNK // G

    @functools.partial(
        pl.kernel,
        out_type=jax.ShapeDtypeStruct((NC * NPAD,), jnp.float32),
        mesh=_MESH(),
        scratch_types=[
            pltpu.VMEM((PT,), jnp.float32),
            pltpu.VMEM((128,), jnp.float32),
            pltpu.VMEM((NCHUNK, K), jnp.int32),
            pltpu.VMEM_SHARED((NPAD,), jnp.float32),
            pltpu.SemaphoreType.DMA,
        ],
    )
    def deg_kernel(e4_hbm, deg_out, zbuf, ones_v, idx_v, hist, sem):
        c = lax.axis_index("c")
        s = lax.axis_index("s")
        w = c * NS + s

        def fill_z(i, _):
            zbuf[pl.ds(i * 16, 16)] = jnp.zeros((16,), jnp.float32)
            return 0

        lax.fori_loop(0, PT // 16, fill_z, 0)
        for j in range(8):
            ones_v[pl.ds(j * 16, 16)] = jnp.ones((16,), jnp.float32)

        pltpu.sync_copy(e4_hbm.at[1, w], idx_v)
        pltpu.sync_copy(zbuf, hist.at[pl.ds(s * PT, PT)])
        plsc.subcore_barrier()

        ones_k = ones_v.at[pl.ds(0, K)]

        def body(gi, _):
            for b in range(G):
                pltpu.async_copy(
                    ones_k, hist.at[idx_v.at[gi * G + b]], sem, add=True
                )
            for b in range(G):
                pltpu.make_async_copy(
                    ones_k, hist.at[idx_v.at[gi * G + b]], sem
                ).wait()
            return 0

        lax.fori_loop(0, NGRP, body, 0)
        plsc.subcore_barrier()
        pltpu.sync_copy(
            hist.at[pl.ds(s * PT, PT)],
            deg_out.at[pl.ds(c * NPAD + s * PT, PT)],
        )

    return deg_kernel


def _agg_kernel(N, NPAD, E, D, K, NCHUNK):
    RPT = N // NS
    CNT = ((RPT + 7) // 8) * 8
    NZ, ZREM = RPT // K, RPT % K
    NBUF = 2
    NPH = 2
    HCH = NCHUNK // NPH
    NGRP = HCH // NBUF

    EPT = E // NW

    @functools.partial(
        pl.kernel,
        out_type=jax.ShapeDtypeStruct((NC * NPAD, D), jnp.float32),
        mesh=_MESH(),
        scratch_types=[
            pltpu.VMEM((HCH, K), jnp.int32),
            pltpu.VMEM((HCH, K), jnp.int32),
            pltpu.VMEM((K, D), jnp.float32),
            pltpu.VMEM((K, D), jnp.float32),
            pltpu.VMEM_SHARED((N, D), jnp.float32),
            pltpu.SemaphoreType.DMA,
            pltpu.SemaphoreType.DMA,
        ],
    )
    def agg_kernel(g_hbm, e4_hbm, s_out,
                   sidx, didx, r0, r1, acc, m0, m1):
        c = lax.axis_index("c")
        s = lax.axis_index("s")
        w = c * NS + s
        rows = (r0, r1)
        sems = (m0, m1)

        def fill_z(i, _):
            for j in range(D // 16):
                r0[i, pl.ds(j * 16, 16)] = jnp.zeros((16,), jnp.float32)
            return 0

        lax.fori_loop(0, K, fill_z, 0)

        row0 = s * RPT
        for k in range(NZ):
            pltpu.sync_copy(r0, acc.at[pl.ds(row0 + k * K, K)])
        if ZREM:
            pltpu.sync_copy(
                r0.at[pl.ds(0, ZREM)], acc.at[pl.ds(row0 + NZ * K, ZREM)]
            )
        plsc.subcore_barrier()

        for ph in range(NPH):
            pltpu.sync_copy(e4_hbm.at[0, w, pl.ds(ph * HCH, HCH)], sidx)
            pltpu.sync_copy(e4_hbm.at[1, w, pl.ds(ph * HCH, HCH)], didx)

            for b in range(NBUF):
                pltpu.async_copy(g_hbm.at[sidx.at[b]], rows[b], sems[b])

            def body(gi, _):
                for b in range(NBUF):
                    i = gi * NBUF + b
                    pltpu.make_async_copy(
                        g_hbm.at[sidx.at[i]], rows[b], sems[b]
                    ).wait()
                    pltpu.sync_copy(rows[b], acc.at[didx.at[i]], add=True)
                    pltpu.async_copy(
                        g_hbm.at[sidx.at[i + NBUF]], rows[b], sems[b]
                    )
                return 0

            lax.fori_loop(0, NGRP - 1, body, 0)
            for b in range(NBUF):
                i = HCH - NBUF + b
                pltpu.make_async_copy(
                    g_hbm.at[sidx.at[i]], rows[b], sems[b]
                ).wait()
                pltpu.sync_copy(rows[b], acc.at[didx.at[i]], add=True)

        plsc.subcore_barrier()

        bs = pl.multiple_of((s * RPT // 8) * 8, 8)
        bs = jnp.minimum(bs, N - CNT)
        pltpu.sync_copy(
            acc.at[pl.ds(bs, CNT)], s_out.at[pl.ds(c * NPAD + bs, CNT)]
        )

    return agg_kernel


def _dinv_col(deg_ref):
    d = deg_ref[0:1, :] + deg_ref[1:2, :] + 1.0
    return jnp.transpose(lax.rsqrt(d), (1, 0))


def _g_body(x_ref, w_ref, deg_ref, g_ref):
    h = lax.dot_general(
        x_ref[...], w_ref[...], (((1,), (1,)), ((), ())),
        preferred_element_type=jnp.float32,
    )
    g_ref[...] = _dinv_col(deg_ref) * h


def _fin_body(s0_ref, s1_ref, g_ref, deg_ref, b_ref, a_ref, o_ref):
    t = s0_ref[...] + s1_ref[...] + g_ref[...]
    t = _dinv_col(deg_ref) * t + b_ref[...]
    o_ref[...] = jnp.where(t >= 0.0, t, a_ref[0, 0] * t)


def kernel(x, edge_index, W, b, a):
    N, D = x.shape
    E = edge_index.shape[1]
    K = 125
    EPT = E // NW
    NCHUNK = EPT // K
    NPAD = ((N + NS * 16 - 1) // (NS * 16)) * (NS * 16)

    e4 = edge_index.reshape(2, NW, NCHUNK, K)

    deg_flat = _deg_kernel(E, NPAD, K, NCHUNK)(e4)
    deg2 = deg_flat.reshape(NC, NPAD)

    RB = 2048
    nb = NPAD // RB
    g = pl.pallas_call(
        _g_body,
        grid=(nb,),
        in_specs=[
            pl.BlockSpec((RB, D), lambda i: (i, 0)),
            pl.BlockSpec((D, D), lambda i: (0, 0)),
            pl.BlockSpec((NC, RB), lambda i: (0, i)),
        ],
        out_specs=pl.BlockSpec((RB, D), lambda i: (i, 0)),
        out_shape=jax.ShapeDtypeStruct((NPAD, D), jnp.float32),
    )(x, W, deg2)

    s_flat = _agg_kernel(N, NPAD, E, D, K, NCHUNK)(g, e4)

    out = pl.pallas_call(
        _fin_body,
        grid=(nb,),
        in_specs=[
            pl.BlockSpec((RB, D), lambda i: (i, 0)),
            pl.BlockSpec((RB, D), lambda i, nb=nb: (i + nb, 0)),
            pl.BlockSpec((RB, D), lambda i: (i, 0)),
            pl.BlockSpec((NC, RB), lambda i: (0, i)),
            pl.BlockSpec((1, D), lambda i: (0, 0)),
            pl.BlockSpec((1, 1), lambda i: (0, 0)),
        ],
        out_specs=pl.BlockSpec((RB, D), lambda i: (i, 0)),
        out_shape=jax.ShapeDtypeStruct((N, D), jnp.float32),
    )(s_flat, s_flat, g, deg2, jnp.reshape(b, (1, D)), jnp.reshape(a, (1, 1)))
    return out
